# Initial kernel scaffold; baseline (speedup 1.0000x reference)
import functools

import jax
import jax.numpy as jnp
from jax import lax
from jax.experimental import pallas as pl
from jax.experimental.pallas import tpu as pltpu
from jax.experimental.pallas import tpu_sc as plsc

N = 50000
D = 256
S = 512
NC = 2
NS = 16
NW = NC * NS
C = 128
B8 = N // 8
NB_BASE = B8 // NW
NB_REM = B8 % NW
K = (8 * (NB_BASE + 1) + C - 1) // C
ACC_ROWS = 16 * 33
ZPR = ACC_ROWS // NS
OPR = S // NS


def _seg_sum_body(h_ref, b_ref, z_ref, out_ref, acc, rows, idxb):
    c = lax.axis_index("c")
    s = lax.axis_index("s")
    wid = s * NC + c

    pltpu.sync_copy(z_ref.at[pl.ds(s * ZPR, ZPR)], acc.at[pl.ds(s * ZPR, ZPR)])
    plsc.subcore_barrier()

    nb = NB_BASE + jnp.where(wid < NB_REM, 1, 0)
    start = (wid * NB_BASE + jnp.minimum(wid, NB_REM)) * 8
    n = nb * 8

    def chunk(i, carry):
        off = jnp.minimum(i * C, n - C)
        base = start + off
        pltpu.sync_copy(b_ref.at[pl.ds(base, C)], idxb)
        dup = i * C - off
        for j in range(C // 16):
            pos = j * 16 + lax.iota(jnp.int32, 16)
            v = idxb[pl.ds(j * 16, 16)]
            idxb[pl.ds(j * 16, 16)] = jnp.where(pos < dup, S, v)
        pltpu.sync_copy(h_ref.at[pl.ds(base, C)], rows)
        pltpu.sync_copy(rows, acc.at[idxb], add=True)
        return carry

    lax.fori_loop(0, K, chunk, 0)
    plsc.subcore_barrier()

    pltpu.sync_copy(acc.at[pl.ds(s * OPR, OPR)],
                    out_ref.at[c, pl.ds(s * OPR, OPR)])


@functools.partial(
    pl.kernel,
    out_type=jax.ShapeDtypeStruct((NC, S, D), jnp.float32),
    mesh=plsc.VectorSubcoreMesh(core_axis_name="c", subcore_axis_name="s"),
    scratch_types=[
        pltpu.VMEM_SHARED((ACC_ROWS, D), jnp.float32),
        pltpu.VMEM((C, D), jnp.float32),
        pltpu.VMEM((C,), jnp.int32),
    ],
)
def _seg_sum_sc(h_ref, b_ref, z_ref, out_ref, acc, rows, idxb):
    _seg_sum_body(h_ref, b_ref, z_ref, out_ref, acc, rows, idxb)


def _mlp_body(p_ref, w1_ref, b1_ref, w2_ref, b2_ref, o_ref):
    hg = p_ref[0] + p_ref[1]
    x = jnp.dot(hg, w1_ref[...], preferred_element_type=jnp.float32)
    x = x + b1_ref[...]
    x = x * jax.nn.sigmoid(x)
    e = jnp.dot(x, w2_ref[...], preferred_element_type=jnp.float32)
    o_ref[...] = e + b2_ref[...]


def kernel(h, batch, W1, b1, W2, b2):
    batch = batch.astype(jnp.int32)
    zeros = jnp.zeros((ACC_ROWS, D), jnp.float32)
    partials = _seg_sum_sc(h, batch, zeros)
    energy = pl.pallas_call(
        _mlp_body,
        out_shape=jax.ShapeDtypeStruct((S, 1), jnp.float32),
    )(partials, W1, b1.reshape(1, -1), W2, b2.reshape(1, -1))
    return energy


# speedup vs baseline: 4.7916x; 4.7916x over previous
import functools

import jax
import jax.numpy as jnp
from jax import lax
from jax.experimental import pallas as pl
from jax.experimental.pallas import tpu as pltpu
from jax.experimental.pallas import tpu_sc as plsc

N = 50000
D = 256
DH = D // 2
S = 512
NC = 2
NS = 16
C = 128
B8 = N // 8
NB_BASE = B8 // NS
NB_REM = B8 % NS
K = (8 * (NB_BASE + 1) + C - 1) // C
ACC_ROWS = 520
OPR = S // NS


def _seg_sum_body(h_ref, b_ref, out_ref, acc, rows, idxb, zbuf):
    c = lax.axis_index("c")
    s = lax.axis_index("s")
    col0 = c * DH

    for r in range(OPR):
        for j in range(DH // 16):
            zbuf[r, pl.ds(j * 16, 16)] = jnp.zeros((16,), jnp.float32)
    pltpu.sync_copy(zbuf, acc.at[pl.ds(s * OPR, OPR)])
    plsc.subcore_barrier()

    nb = NB_BASE + jnp.where(s < NB_REM, 1, 0)
    start = (s * NB_BASE + jnp.minimum(s, NB_REM)) * 8
    n = nb * 8

    def chunk(i, carry):
        off = jnp.minimum(i * C, n - C)
        base = start + off
        pltpu.sync_copy(b_ref.at[pl.ds(base, C)], idxb)
        dup = i * C - off
        for j in range(C // 16):
            pos = j * 16 + lax.iota(jnp.int32, 16)
            v = idxb[pl.ds(j * 16, 16)]
            idxb[pl.ds(j * 16, 16)] = jnp.where(pos < dup, S, v)
        pltpu.sync_copy(h_ref.at[pl.ds(base, C), pl.ds(col0, DH)], rows)
        pltpu.sync_copy(rows, acc.at[idxb], add=True)
        return carry

    lax.fori_loop(0, K, chunk, 0)
    plsc.subcore_barrier()

    pltpu.sync_copy(acc.at[pl.ds(s * OPR, OPR)],
                    out_ref.at[pl.ds(s * OPR, OPR), pl.ds(col0, DH)])


@functools.partial(
    pl.kernel,
    out_type=jax.ShapeDtypeStruct((S, D), jnp.float32),
    mesh=plsc.VectorSubcoreMesh(core_axis_name="c", subcore_axis_name="s"),
    scratch_types=[
        pltpu.VMEM_SHARED((ACC_ROWS, DH), jnp.float32),
        pltpu.VMEM((C, DH), jnp.float32),
        pltpu.VMEM((C,), jnp.int32),
        pltpu.VMEM((OPR, DH), jnp.float32),
    ],
)
def _seg_sum_sc(h_ref, b_ref, out_ref, acc, rows, idxb, zbuf):
    _seg_sum_body(h_ref, b_ref, out_ref, acc, rows, idxb, zbuf)


def _mlp_body(p_ref, w1_ref, b1_ref, w2_ref, b2_ref, o_ref):
    x = jnp.dot(p_ref[...], w1_ref[...], preferred_element_type=jnp.float32)
    x = x + b1_ref[...]
    x = x * jax.nn.sigmoid(x)
    e = jnp.dot(x, w2_ref[...], preferred_element_type=jnp.float32)
    o_ref[...] = e + b2_ref[...]


def kernel(h, batch, W1, b1, W2, b2):
    batch = batch.astype(jnp.int32)
    pooled = _seg_sum_sc(h, batch)
    energy = pl.pallas_call(
        _mlp_body,
        out_shape=jax.ShapeDtypeStruct((S, 1), jnp.float32),
    )(pooled, W1, b1.reshape(1, -1), W2, b2.reshape(1, -1))
    return energy


# speedup vs baseline: 6.7765x; 1.4142x over previous
import functools

import jax
import jax.numpy as jnp
from jax import lax
from jax.experimental import pallas as pl
from jax.experimental.pallas import tpu as pltpu
from jax.experimental.pallas import tpu_sc as plsc

N = 50000
D = 256
DH = D // 2
S = 512
NC = 2
NS = 16
C = 128
B8 = N // 8
NB_BASE = B8 // NS
NB_REM = B8 % NS
K = (8 * (NB_BASE + 1) + C - 1) // C
ACC_ROWS = 520
OPR = S // NS


def _seg_sum_body(h_ref, b_ref, out_ref, acc,
                  rows0, rows1, idx0, idx1, zbuf,
                  semr0, semr1, semi0, semi1):
    c = lax.axis_index("c")
    s = lax.axis_index("s")
    col0 = c * DH
    rows_b = (rows0, rows1)
    idx_b = (idx0, idx1)
    semr = (semr0, semr1)
    semi = (semi0, semi1)

    for r in range(OPR):
        for j in range(DH // 16):
            zbuf[r, pl.ds(j * 16, 16)] = jnp.zeros((16,), jnp.float32)
    pltpu.sync_copy(zbuf, acc.at[pl.ds(s * OPR, OPR)])
    plsc.subcore_barrier()

    nb = NB_BASE + jnp.where(s < NB_REM, 1, 0)
    start = (s * NB_BASE + jnp.minimum(s, NB_REM)) * 8
    n = nb * 8

    def chunk_base(i):
        return start + jnp.minimum(i * C, n - C)

    def start_dma(i, slot):
        base = chunk_base(i)
        pltpu.async_copy(b_ref.at[pl.ds(base, C)], idx_b[slot], semi[slot])
        pltpu.async_copy(h_ref.at[pl.ds(base, C), pl.ds(col0, DH)],
                         rows_b[slot], semr[slot])

    def wait_dma(i, slot):
        base = chunk_base(i)
        pltpu.make_async_copy(b_ref.at[pl.ds(base, C)],
                              idx_b[slot], semi[slot]).wait()
        pltpu.make_async_copy(h_ref.at[pl.ds(base, C), pl.ds(col0, DH)],
                              rows_b[slot], semr[slot]).wait()

    start_dma(0, 0)
    start_dma(1, 1)
    for i in range(K):
        slot = i % 2
        wait_dma(i, slot)
        if i == K - 1:
            dup = (K - 1) * C - (jnp.minimum((K - 1) * C, n - C))
            for j in range(C // 16):
                pos = j * 16 + lax.iota(jnp.int32, 16)
                v = idx_b[slot][pl.ds(j * 16, 16)]
                idx_b[slot][pl.ds(j * 16, 16)] = jnp.where(pos < dup, S, v)
        pltpu.sync_copy(rows_b[slot], acc.at[idx_b[slot]], add=True)
        if i + 2 < K:
            start_dma(i + 2, slot)
    plsc.subcore_barrier()

    pltpu.sync_copy(acc.at[pl.ds(s * OPR, OPR)],
                    out_ref.at[pl.ds(s * OPR, OPR), pl.ds(col0, DH)])


@functools.partial(
    pl.kernel,
    out_type=jax.ShapeDtypeStruct((S, D), jnp.float32),
    mesh=plsc.VectorSubcoreMesh(core_axis_name="c", subcore_axis_name="s"),
    scratch_types=[
        pltpu.VMEM_SHARED((ACC_ROWS, DH), jnp.float32),
        pltpu.VMEM((C, DH), jnp.float32),
        pltpu.VMEM((C, DH), jnp.float32),
        pltpu.VMEM((C,), jnp.int32),
        pltpu.VMEM((C,), jnp.int32),
        pltpu.VMEM((OPR, DH), jnp.float32),
        pltpu.SemaphoreType.DMA,
        pltpu.SemaphoreType.DMA,
        pltpu.SemaphoreType.DMA,
        pltpu.SemaphoreType.DMA,
    ],
)
def _seg_sum_sc(h_ref, b_ref, out_ref, acc, rows0, rows1, idx0, idx1, zbuf,
                semr0, semr1, semi0, semi1):
    _seg_sum_body(h_ref, b_ref, out_ref, acc, rows0, rows1, idx0, idx1, zbuf,
                  semr0, semr1, semi0, semi1)


def _mlp_body(p_ref, w1_ref, b1_ref, w2_ref, b2_ref, o_ref):
    x = jnp.dot(p_ref[...], w1_ref[...], preferred_element_type=jnp.float32)
    x = x + b1_ref[...]
    x = x * jax.nn.sigmoid(x)
    e = jnp.dot(x, w2_ref[...], preferred_element_type=jnp.float32)
    o_ref[...] = e + b2_ref[...]


def kernel(h, batch, W1, b1, W2, b2):
    batch = batch.astype(jnp.int32)
    pooled = _seg_sum_sc(h, batch)
    energy = pl.pallas_call(
        _mlp_body,
        out_shape=jax.ShapeDtypeStruct((S, 1), jnp.float32),
    )(pooled, W1, b1.reshape(1, -1), W2, b2.reshape(1, -1))
    return energy
